# baseline (speedup 1.0000x reference)
import functools

import jax
import jax.numpy as jnp
from jax import lax
from jax.experimental import pallas as pl
from jax.experimental.pallas import tpu as pltpu
from jax.experimental.pallas import tpu_sc as plsc

_N = 10000
_D = 128
_E = 320000

_NC = 2
_NS = 16
_NW = _NC * _NS

_CH = 128
_GRP = 16
_CPT = 80
_NCH = _NW * _CPT
_EPAD = _NCH * _CH

_R = 10240
_ZPT = _R // _NS


def _sc_aggregate_body(src_hbm, dst_hbm, h_hbm, out_hbm,
                       src_v, dst_v, rows_a, rows_b, agg_sh, sem_a, sem_b):
    cid = lax.axis_index("c")
    sid = lax.axis_index("s")
    wid = sid * _NC + cid
    tile_base = wid * _CPT

    zero16 = jnp.zeros((16,), jnp.float32)

    with jax.named_scope("sc_zero"):
        def zrow(i, c):
            rows_a[i // (_D // 16), pl.ds((i % (_D // 16)) * 16, 16)] = zero16
            return c

        lax.fori_loop(0, _CH * (_D // 16), zrow, 0)
        for k in range(_ZPT // _CH):
            pltpu.sync_copy(rows_a, agg_sh.at[pl.ds(sid * _ZPT + k * _CH, _CH)])
        plsc.subcore_barrier()

    def wait_gather(buf, sem):
        pltpu.make_async_copy(h_hbm.at[src_v.at[0]], buf, sem).wait()

    def group(g, carry):
        base = tile_base + g * _GRP
        pltpu.sync_copy(src_hbm.at[pl.ds(base, _GRP)], src_v)
        pltpu.sync_copy(dst_hbm.at[pl.ds(base, _GRP)], dst_v)
        pltpu.async_copy(h_hbm.at[src_v.at[0]], rows_a, sem_a)

        def steady(jj, c):
            j0 = 2 * jj
            pltpu.async_copy(h_hbm.at[src_v.at[j0 + 1]], rows_b, sem_b)
            wait_gather(rows_a, sem_a)
            pltpu.sync_copy(rows_a, agg_sh.at[dst_v.at[j0]], add=True)

            @pl.when(jj < _GRP // 2 - 1)
            def _():
                pltpu.async_copy(h_hbm.at[src_v.at[j0 + 2]], rows_a, sem_a)

            wait_gather(rows_b, sem_b)
            pltpu.sync_copy(rows_b, agg_sh.at[dst_v.at[j0 + 1]], add=True)
            return c

        lax.fori_loop(0, _GRP // 2, steady, 0)
        return carry

    with jax.named_scope("sc_agg"):
        lax.fori_loop(0, _CPT // _GRP, group, 0)
        plsc.subcore_barrier()

    with jax.named_scope("sc_out"):
        pltpu.sync_copy(agg_sh.at[pl.ds(sid * _ZPT, _ZPT)],
                        out_hbm.at[pl.ds(cid * _R + sid * _ZPT, _ZPT)])


def _sc_aggregate(src_p, dst_p, h):
    mesh = plsc.VectorSubcoreMesh(core_axis_name="c", subcore_axis_name="s")
    kern = pl.kernel(
        _sc_aggregate_body,
        mesh=mesh,
        out_type=jax.ShapeDtypeStruct((_NC * _R, _D), jnp.float32),
        scratch_types=[
            pltpu.VMEM((_GRP, _CH), jnp.int32),
            pltpu.VMEM((_GRP, _CH), jnp.int32),
            pltpu.VMEM((_CH, _D), jnp.float32),
            pltpu.VMEM((_CH, _D), jnp.float32),
            pltpu.VMEM_SHARED((_R, _D), jnp.float32),
            pltpu.SemaphoreType.DMA,
            pltpu.SemaphoreType.DMA,
        ],
    )
    return kern(src_p, dst_p, h)


def _tc_dense_body(agg0_ref, agg1_ref, h_ref, wc_ref, wih_ref, whh_ref,
                   bc_ref, bih_ref, bhh_ref, gamma_ref, beta_ref, out_ref):
    f32 = jnp.float32
    agg = agg0_ref[:] + agg1_ref[:]
    h = h_ref[:]
    hp = jnp.dot(agg, wc_ref[:], preferred_element_type=f32) + bc_ref[:]
    hp = jnp.where(hp > 0, hp, jnp.exp(jnp.minimum(hp, 0.0)) - 1.0)
    gi = jnp.dot(hp, wih_ref[:], preferred_element_type=f32) + bih_ref[:]
    gh = jnp.dot(h, whh_ref[:], preferred_element_type=f32) + bhh_ref[:]
    r = jax.nn.sigmoid(gi[:, :_D] + gh[:, :_D])
    z = jax.nn.sigmoid(gi[:, _D:2 * _D] + gh[:, _D:2 * _D])
    n = jnp.tanh(gi[:, 2 * _D:] + r * gh[:, 2 * _D:])
    h_new = jnp.maximum((1.0 - z) * n + z * h, 0.0)
    mu = jnp.mean(h_new, axis=1, keepdims=True)
    c = h_new - mu
    var = jnp.mean(c * c, axis=1, keepdims=True)
    out_ref[:] = gamma_ref[:] * c * lax.rsqrt(var + 1e-5) + beta_ref[:]


def _tc_dense(agg0, agg1, h, wc_t, wih_t, whh_t, bc, bih, bhh, gamma, beta):
    blk = 1000
    grid = _N // blk
    row_spec = pl.BlockSpec((blk, _D), lambda i: (i, 0))
    full = lambda shape: pl.BlockSpec(shape, lambda i: (0, 0))
    return pl.pallas_call(
        _tc_dense_body,
        grid=(grid,),
        in_specs=[
            row_spec, row_spec, row_spec,
            full((_D, _D)), full((_D, 3 * _D)), full((_D, 3 * _D)),
            full((1, _D)), full((1, 3 * _D)), full((1, 3 * _D)),
            full((1, _D)), full((1, _D)),
        ],
        out_specs=row_spec,
        out_shape=jax.ShapeDtypeStruct((_N, _D), jnp.float32),
    )(agg0, agg1, h, wc_t, wih_t, whh_t, bc, bih, bhh, gamma, beta)


def kernel(h, edge_index, W_conv, b_conv, W_ih, W_hh, b_ih, b_hh, gamma, beta):
    src = edge_index[0]
    dst = edge_index[1]
    pad = _EPAD - _E
    src_p = jnp.concatenate(
        [src, jnp.zeros((pad,), jnp.int32)]).reshape(_NCH, _CH)
    dst_p = jnp.concatenate(
        [dst, jnp.full((pad,), _N, jnp.int32)]).reshape(_NCH, _CH)
    parts = _sc_aggregate(src_p, dst_p, h)
    agg0 = parts[:_N]
    agg1 = parts[_R:_R + _N]

    return _tc_dense(
        agg0, agg1, h,
        W_conv.T, W_ih.T, W_hh.T,
        b_conv.reshape(1, _D), b_ih.reshape(1, 3 * _D),
        b_hh.reshape(1, 3 * _D),
        gamma.reshape(1, _D), beta.reshape(1, _D),
    )


# speedup vs baseline: 3.0893x; 3.0893x over previous
import functools

import jax
import jax.numpy as jnp
from jax import lax
from jax.experimental import pallas as pl
from jax.experimental.pallas import tpu as pltpu
from jax.experimental.pallas import tpu_sc as plsc

_N = 10000
_D = 128
_E = 320000

_NC = 2
_NS = 16
_NW = _NC * _NS

_CH = 128
_GRP = 16
_CPT = 80
_NCH = _NW * _CPT
_EPAD = _NCH * _CH

_R = 10240
_ZPT = _R // _NS


def _sc_aggregate_body(src_hbm, dst_hbm, h_hbm, out_hbm,
                       src_v, dst_v, rows_a, rows_b, agg_sh, sem_a, sem_b):
    cid = lax.axis_index("c")
    sid = lax.axis_index("s")
    wid = sid * _NC + cid
    tile_base = wid * _CPT

    zero16 = jnp.zeros((16,), jnp.float32)

    with jax.named_scope("sc_zero"):
        def zrow(i, c):
            rows_a[i // (_D // 16), pl.ds((i % (_D // 16)) * 16, 16)] = zero16
            return c

        lax.fori_loop(0, _CH * (_D // 16), zrow, 0)
        for k in range(_ZPT // _CH):
            pltpu.sync_copy(rows_a, agg_sh.at[pl.ds(sid * _ZPT + k * _CH, _CH)])
        plsc.subcore_barrier()

    def wait_gather(buf, sem):
        pltpu.make_async_copy(h_hbm.at[src_v.at[0]], buf, sem).wait()

    def group(g, carry):
        base = tile_base + g * _GRP
        pltpu.sync_copy(src_hbm.at[pl.ds(base, _GRP)], src_v)
        pltpu.sync_copy(dst_hbm.at[pl.ds(base, _GRP)], dst_v)
        pltpu.async_copy(h_hbm.at[src_v.at[0]], rows_a, sem_a)

        def steady(jj, c):
            j0 = 2 * jj
            pltpu.async_copy(h_hbm.at[src_v.at[j0 + 1]], rows_b, sem_b)
            wait_gather(rows_a, sem_a)
            pltpu.sync_copy(rows_a, agg_sh.at[dst_v.at[j0]], add=True)

            @pl.when(jj < _GRP // 2 - 1)
            def _():
                pltpu.async_copy(h_hbm.at[src_v.at[j0 + 2]], rows_a, sem_a)

            wait_gather(rows_b, sem_b)
            pltpu.sync_copy(rows_b, agg_sh.at[dst_v.at[j0 + 1]], add=True)
            return c

        lax.fori_loop(0, _GRP // 2, steady, 0)
        return carry

    with jax.named_scope("sc_agg"):
        lax.fori_loop(0, _CPT // _GRP, group, 0)
        plsc.subcore_barrier()

    with jax.named_scope("sc_out"):
        pltpu.sync_copy(agg_sh.at[pl.ds(sid * _ZPT, _ZPT)],
                        out_hbm.at[pl.ds(cid * _R + sid * _ZPT, _ZPT)])


def _sc_aggregate(src_p, dst_p, h):
    mesh = plsc.VectorSubcoreMesh(core_axis_name="c", subcore_axis_name="s")
    kern = pl.kernel(
        _sc_aggregate_body,
        mesh=mesh,
        out_type=jax.ShapeDtypeStruct((_NC * _R, _D), jnp.float32),
        scratch_types=[
            pltpu.VMEM((_GRP, _CH), jnp.int32),
            pltpu.VMEM((_GRP, _CH), jnp.int32),
            pltpu.VMEM((_CH, _D), jnp.float32),
            pltpu.VMEM((_CH, _D), jnp.float32),
            pltpu.VMEM_SHARED((_R, _D), jnp.float32),
            pltpu.SemaphoreType.DMA,
            pltpu.SemaphoreType.DMA,
        ],
    )
    return kern(src_p, dst_p, h)


def _tc_dense_body(agg0_ref, agg1_ref, h_ref, wc_ref, wih_ref, whh_ref,
                   bc_ref, bih_ref, bhh_ref, gamma_ref, beta_ref, out_ref):
    f32 = jnp.float32
    agg = agg0_ref[:] + agg1_ref[:]
    h = h_ref[:]
    hp = jnp.dot(agg, wc_ref[:], preferred_element_type=f32) + bc_ref[:]
    hp = jnp.where(hp > 0, hp, jnp.exp(jnp.minimum(hp, 0.0)) - 1.0)
    gi = jnp.dot(hp, wih_ref[:], preferred_element_type=f32) + bih_ref[:]
    gh = jnp.dot(h, whh_ref[:], preferred_element_type=f32) + bhh_ref[:]
    r = jax.nn.sigmoid(gi[:, :_D] + gh[:, :_D])
    z = jax.nn.sigmoid(gi[:, _D:2 * _D] + gh[:, _D:2 * _D])
    n = jnp.tanh(gi[:, 2 * _D:] + r * gh[:, 2 * _D:])
    h_new = jnp.maximum((1.0 - z) * n + z * h, 0.0)
    mu = jnp.mean(h_new, axis=1, keepdims=True)
    c = h_new - mu
    var = jnp.mean(c * c, axis=1, keepdims=True)
    out_ref[:] = gamma_ref[:] * c * lax.rsqrt(var + 1e-5) + beta_ref[:]


def _tc_dense(agg0, agg1, h, wc_t, wih_t, whh_t, bc, bih, bhh, gamma, beta):
    blk = 1000
    grid = _N // blk
    row_spec = pl.BlockSpec((blk, _D), lambda i: (i, 0))
    full = lambda shape: pl.BlockSpec(shape, lambda i: (0, 0))
    return pl.pallas_call(
        _tc_dense_body,
        grid=(grid,),
        in_specs=[
            row_spec, row_spec, row_spec,
            full((_D, _D)), full((_D, 3 * _D)), full((_D, 3 * _D)),
            full((1, _D)), full((1, 3 * _D)), full((1, 3 * _D)),
            full((1, _D)), full((1, _D)),
        ],
        out_specs=row_spec,
        out_shape=jax.ShapeDtypeStruct((_N, _D), jnp.float32),
    )(agg0, agg1, h, wc_t, wih_t, whh_t, bc, bih, bhh, gamma, beta)


def kernel(h, edge_index, W_conv, b_conv, W_ih, W_hh, b_ih, b_hh, gamma, beta):
    src = edge_index[0]
    dst = edge_index[1]
    pad = _EPAD - _E
    pad_src = (jnp.arange(pad, dtype=jnp.int32) * 7) % _N
    pad_dst = _N + (jnp.arange(pad, dtype=jnp.int32) % (_R - _N))
    src_p = jnp.concatenate([src, pad_src]).reshape(_NCH, _CH)
    dst_p = jnp.concatenate([dst, pad_dst]).reshape(_NCH, _CH)
    parts = _sc_aggregate(src_p, dst_p, h)
    agg0 = parts[:_N]
    agg1 = parts[_R:_R + _N]

    return _tc_dense(
        agg0, agg1, h,
        W_conv.T, W_ih.T, W_hh.T,
        b_conv.reshape(1, _D), b_ih.reshape(1, 3 * _D),
        b_hh.reshape(1, 3 * _D),
        gamma.reshape(1, _D), beta.reshape(1, _D),
    )


# speedup vs baseline: 3.2947x; 1.0665x over previous
import functools

import jax
import jax.numpy as jnp
from jax import lax
from jax.experimental import pallas as pl
from jax.experimental.pallas import tpu as pltpu
from jax.experimental.pallas import tpu_sc as plsc

_N = 10000
_D = 128
_E = 320000

_NC = 2
_NS = 16
_NW = _NC * _NS

_CH = 128
_GRP = 16
_CPT = 80
_NCH = _E // _CH
_TAIL = _NCH - (_NW - 1) * _CPT

_R = 10240
_ZPT = _R // _NS


def _sc_aggregate_body(ei_hbm, h_hbm, out_hbm,
                       src_v, dst_v, rows_a, rows_b, agg_sh, sem_a, sem_b):
    cid = lax.axis_index("c")
    sid = lax.axis_index("s")
    wid = sid * _NC + cid
    tile_base = wid * _CPT
    last = wid == _NW - 1

    zero16 = jnp.zeros((16,), jnp.float32)

    with jax.named_scope("sc_zero"):
        def zrow(i, c):
            rows_a[i // (_D // 16), pl.ds((i % (_D // 16)) * 16, 16)] = zero16
            return c

        lax.fori_loop(0, _CH * (_D // 16), zrow, 0)
        for k in range(_ZPT // _CH):
            pltpu.sync_copy(rows_a, agg_sh.at[pl.ds(sid * _ZPT + k * _CH, _CH)])
        plsc.subcore_barrier()

    def wait_gather(buf, sem):
        pltpu.make_async_copy(h_hbm.at[src_v.at[0]], buf, sem).wait()

    def pipeline(npairs):
        pltpu.async_copy(h_hbm.at[src_v.at[0]], rows_a, sem_a)

        def steady(jj, c):
            j0 = 2 * jj
            pltpu.async_copy(h_hbm.at[src_v.at[j0 + 1]], rows_b, sem_b)
            wait_gather(rows_a, sem_a)
            pltpu.sync_copy(rows_a, agg_sh.at[dst_v.at[j0]], add=True)

            @pl.when(jj < npairs - 1)
            def _():
                pltpu.async_copy(h_hbm.at[src_v.at[j0 + 2]], rows_a, sem_a)

            wait_gather(rows_b, sem_b)
            pltpu.sync_copy(rows_b, agg_sh.at[dst_v.at[j0 + 1]], add=True)
            return c

        lax.fori_loop(0, npairs, steady, 0)

    def group(g, carry):
        base = tile_base + g * _GRP
        pltpu.sync_copy(ei_hbm.at[0, pl.ds(base, _GRP)], src_v)
        pltpu.sync_copy(ei_hbm.at[1, pl.ds(base, _GRP)], dst_v)
        pipeline(_GRP // 2)
        return carry

    with jax.named_scope("sc_agg"):
        lax.fori_loop(0, jnp.where(last, _TAIL // _GRP, _CPT // _GRP),
                      group, 0)

        @pl.when(last)
        def _():
            base = tile_base + (_TAIL // _GRP) * _GRP
            t = _TAIL % _GRP
            pltpu.sync_copy(ei_hbm.at[0, pl.ds(base, t)],
                            src_v.at[pl.ds(0, t)])
            pltpu.sync_copy(ei_hbm.at[1, pl.ds(base, t)],
                            dst_v.at[pl.ds(0, t)])
            pipeline(t // 2)

        plsc.subcore_barrier()

    with jax.named_scope("sc_out"):
        pltpu.sync_copy(agg_sh.at[pl.ds(sid * _ZPT, _ZPT)],
                        out_hbm.at[pl.ds(cid * _R + sid * _ZPT, _ZPT)])


def _sc_aggregate(ei3, h):
    mesh = plsc.VectorSubcoreMesh(core_axis_name="c", subcore_axis_name="s")
    kern = pl.kernel(
        _sc_aggregate_body,
        mesh=mesh,
        out_type=jax.ShapeDtypeStruct((_NC * _R, _D), jnp.float32),
        scratch_types=[
            pltpu.VMEM((_GRP, _CH), jnp.int32),
            pltpu.VMEM((_GRP, _CH), jnp.int32),
            pltpu.VMEM((_CH, _D), jnp.float32),
            pltpu.VMEM((_CH, _D), jnp.float32),
            pltpu.VMEM_SHARED((_R, _D), jnp.float32),
            pltpu.SemaphoreType.DMA,
            pltpu.SemaphoreType.DMA,
        ],
    )
    return kern(ei3, h)


def _tc_dense_body(agg0_ref, agg1_ref, h_ref, wc_ref, wih_ref, whh_ref,
                   bc_ref, bih_ref, bhh_ref, gamma_ref, beta_ref, out_ref):
    f32 = jnp.float32
    agg = agg0_ref[:] + agg1_ref[:]
    h = h_ref[:]
    hp = jnp.dot(agg, wc_ref[:], preferred_element_type=f32) + bc_ref[:]
    hp = jnp.where(hp > 0, hp, jnp.exp(jnp.minimum(hp, 0.0)) - 1.0)
    gi = jnp.dot(hp, wih_ref[:], preferred_element_type=f32) + bih_ref[:]
    gh = jnp.dot(h, whh_ref[:], preferred_element_type=f32) + bhh_ref[:]
    r = jax.nn.sigmoid(gi[:, :_D] + gh[:, :_D])
    z = jax.nn.sigmoid(gi[:, _D:2 * _D] + gh[:, _D:2 * _D])
    n = jnp.tanh(gi[:, 2 * _D:] + r * gh[:, 2 * _D:])
    h_new = jnp.maximum((1.0 - z) * n + z * h, 0.0)
    mu = jnp.mean(h_new, axis=1, keepdims=True)
    c = h_new - mu
    var = jnp.mean(c * c, axis=1, keepdims=True)
    out_ref[:] = gamma_ref[:] * c * lax.rsqrt(var + 1e-5) + beta_ref[:]


def _tc_dense(agg0, agg1, h, wc_t, wih_t, whh_t, bc, bih, bhh, gamma, beta):
    blk = 1000
    grid = _N // blk
    row_spec = pl.BlockSpec((blk, _D), lambda i: (i, 0))
    full = lambda shape: pl.BlockSpec(shape, lambda i: (0, 0))
    return pl.pallas_call(
        _tc_dense_body,
        grid=(grid,),
        in_specs=[
            row_spec, row_spec, row_spec,
            full((_D, _D)), full((_D, 3 * _D)), full((_D, 3 * _D)),
            full((1, _D)), full((1, 3 * _D)), full((1, 3 * _D)),
            full((1, _D)), full((1, _D)),
        ],
        out_specs=row_spec,
        out_shape=jax.ShapeDtypeStruct((_N, _D), jnp.float32),
    )(agg0, agg1, h, wc_t, wih_t, whh_t, bc, bih, bhh, gamma, beta)


def kernel(h, edge_index, W_conv, b_conv, W_ih, W_hh, b_ih, b_hh, gamma, beta):
    ei3 = edge_index.reshape(2, _NCH, _CH)
    parts = _sc_aggregate(ei3, h)
    agg0 = parts[:_N]
    agg1 = parts[_R:_R + _N]

    return _tc_dense(
        agg0, agg1, h,
        W_conv.T, W_ih.T, W_hh.T,
        b_conv.reshape(1, _D), b_ih.reshape(1, 3 * _D),
        b_hh.reshape(1, 3 * _D),
        gamma.reshape(1, _D), beta.reshape(1, _D),
    )
